# baseline (speedup 1.0000x reference)
import functools

import jax
import jax.numpy as jnp
from jax.experimental import pallas as pl

B, C, N = 2, 6, 24000
K = 10
S = 3072
H = 256

NP = 24576
BLK1 = 3072


def _stage1_body(pts_ref, w1_ref, b1_ref, wh_ref, head_ref, ptst_ref):
    x = pts_ref[0]
    xt = x.T
    feat = jax.nn.relu(
        jax.lax.dot_general(xt, w1_ref[...], (((1,), (0,)), ((), ())),
                            preferred_element_type=jnp.float32)
        + b1_ref[...][None, :]
    )
    head = jax.lax.dot_general(feat, wh_ref[...], (((1,), (0,)), ((), ())),
                               preferred_element_type=jnp.float32)
    head_ref[0] = head
    ptst_ref[0] = jnp.pad(xt, ((0, 0), (0, 16 - C)))


def _stage1(points, W1, b1, Whead):
    points = jnp.pad(points, ((0, 0), (0, 0), (0, NP - N)))
    nb = NP // BLK1
    head, pts_t = pl.pallas_call(
        _stage1_body,
        grid=(B, nb),
        in_specs=[
            pl.BlockSpec((1, C, BLK1), lambda b, j: (b, 0, j)),
            pl.BlockSpec((C, H), lambda b, j: (0, 0)),
            pl.BlockSpec((H,), lambda b, j: (0,)),
            pl.BlockSpec((H, 16), lambda b, j: (0, 0)),
        ],
        out_specs=[
            pl.BlockSpec((1, BLK1, 16), lambda b, j: (b, j, 0)),
            pl.BlockSpec((1, BLK1, 16), lambda b, j: (b, j, 0)),
        ],
        out_shape=[
            jax.ShapeDtypeStruct((B, NP, 16), jnp.float32),
            jax.ShapeDtypeStruct((B, NP, 16), jnp.float32),
        ],
    )(points, W1, b1, Whead)
    return head[:, :N], pts_t


def _stage2_body(crop_ref, w2_ref, b2_ref, wh_ref, head_ref, cent_ref):
    xt = crop_ref[0, :, :C]
    xyz = xt[:, :3]
    mean = jnp.sum(xyz, axis=0, keepdims=True) / S
    ctr = jnp.concatenate([xyz - mean, xt[:, 3:]], axis=1)
    feat = jax.nn.relu(
        jax.lax.dot_general(ctr, w2_ref[...], (((1,), (0,)), ((), ())),
                            preferred_element_type=jnp.float32)
        + b2_ref[...][None, :]
    )
    head = jax.lax.dot_general(feat, wh_ref[...], (((1,), (0,)), ((), ())),
                               preferred_element_type=jnp.float32)
    head_ref[0] = head
    cent_ref[0] = ctr.T


def _stage2(cropped_t, W2, b2, Whead):
    head, centered = pl.pallas_call(
        _stage2_body,
        grid=(B * K,),
        in_specs=[
            pl.BlockSpec((1, S, 16), lambda i: (i, 0, 0)),
            pl.BlockSpec((C, H), lambda i: (0, 0)),
            pl.BlockSpec((H,), lambda i: (0,)),
            pl.BlockSpec((H, 8), lambda i: (0, 0)),
        ],
        out_specs=[
            pl.BlockSpec((1, S, 8), lambda i: (i, 0, 0)),
            pl.BlockSpec((1, C, S), lambda i: (i, 0, 0)),
        ],
        out_shape=[
            jax.ShapeDtypeStruct((B * K, S, 8), jnp.float32),
            jax.ShapeDtypeStruct((B * K, C, S), jnp.float32),
        ],
    )(cropped_t, W2, b2, Whead)
    return head, centered


def kernel(points, labels, W1, b1, Wsem1, Woff1, Wmask1, W2, b2, Wsem2, Woff2, Wmask2):
    Whead1 = jnp.pad(jnp.concatenate([Wsem1, Woff1, Wmask1], axis=1),
                     ((0, 0), (0, 2)))
    Whead2 = jnp.pad(jnp.concatenate([Wsem2, Woff2, Wmask2], axis=1),
                     ((0, 0), (0, 2)))

    head1, pts_t = _stage1(points, W1, b1, Whead1)
    sem1 = head1[:, :, :K]
    off1 = head1[:, :, K:K + 3]
    mask1 = head1[:, :, K + 3:K + 4]

    coords = jnp.swapaxes(points[:, :3, :], 1, 2)
    lab = labels[:, 0, :]

    def centroids_b(cb, lb):
        s = jax.ops.segment_sum(cb, lb, num_segments=K)
        cnt = jax.ops.segment_sum(jnp.ones((cb.shape[0],), jnp.float32), lb,
                                  num_segments=K)
        return s / jnp.maximum(cnt, 1.0)[:, None]

    cents = jax.vmap(centroids_b)(coords, lab)

    d2 = jnp.sum((coords[:, None, :, :] - cents[:, :, None, :]) ** 2, axis=-1)
    _, idx = jax.lax.top_k(-d2, S)

    flat_idx = (idx + (jnp.arange(B, dtype=idx.dtype) * NP)[:, None, None]
                ).reshape(B * K * S)
    cropped_t = pts_t.reshape(B * NP, 16)[flat_idx].reshape(B * K, S, 16)

    head2, centered = _stage2(cropped_t, W2, b2, Whead2)
    sem2 = head2[:, :, :2]
    off2 = head2[:, :, 2:5]
    mask2 = head2[:, :, 5:6]
    centered = centered.reshape(B, K, C, S)
    return (sem1, off1, mask1, sem2, off2, mask2, centered)


# speedup vs baseline: 1.0857x; 1.0857x over previous
import functools

import jax
import jax.numpy as jnp
from jax.experimental import pallas as pl

B, C, N = 2, 6, 24000
K = 10
S = 3072
H = 256

NP = 24576
BLK1 = 3072


def _stage1_body(pts_ref, w1t_ref, b1_ref, wht_ref, head_ref):
    x = pts_ref[0]
    feat = jax.nn.relu(
        jax.lax.dot_general(w1t_ref[...], x, (((1,), (0,)), ((), ())),
                            preferred_element_type=jnp.float32)
        + b1_ref[...]
    )
    head_ref[0] = jax.lax.dot_general(wht_ref[...], feat,
                                      (((1,), (0,)), ((), ())),
                                      preferred_element_type=jnp.float32)


def _stage1(points, W1t, b1, Wheadt):
    nb = NP // BLK1
    head = pl.pallas_call(
        _stage1_body,
        grid=(B, nb),
        in_specs=[
            pl.BlockSpec((1, C, BLK1), lambda b, j: (b, 0, j)),
            pl.BlockSpec((H, C), lambda b, j: (0, 0)),
            pl.BlockSpec((H, 1), lambda b, j: (0, 0)),
            pl.BlockSpec((16, H), lambda b, j: (0, 0)),
        ],
        out_specs=pl.BlockSpec((1, 16, BLK1), lambda b, j: (b, 0, j)),
        out_shape=jax.ShapeDtypeStruct((B, 16, NP), jnp.float32),
    )(points, W1t, b1, Wheadt)
    return head


def _stage2_body(crop_ref, w2t_ref, b2_ref, wht_ref, head_ref, cent_ref):
    x = crop_ref[0]
    xyz = x[:3, :]
    mean = jnp.sum(xyz, axis=1, keepdims=True) / S
    ctr = jnp.concatenate([xyz - mean, x[3:, :]], axis=0)
    feat = jax.nn.relu(
        jax.lax.dot_general(w2t_ref[...], ctr, (((1,), (0,)), ((), ())),
                            preferred_element_type=jnp.float32)
        + b2_ref[...]
    )
    head_ref[0] = jax.lax.dot_general(wht_ref[...], feat,
                                      (((1,), (0,)), ((), ())),
                                      preferred_element_type=jnp.float32)
    cent_ref[0] = ctr


def _stage2(cropped, W2t, b2, Wheadt):
    head, centered = pl.pallas_call(
        _stage2_body,
        grid=(B * K,),
        in_specs=[
            pl.BlockSpec((1, C, S), lambda i: (i, 0, 0)),
            pl.BlockSpec((H, C), lambda i: (0, 0)),
            pl.BlockSpec((H, 1), lambda i: (0, 0)),
            pl.BlockSpec((8, H), lambda i: (0, 0)),
        ],
        out_specs=[
            pl.BlockSpec((1, 8, S), lambda i: (i, 0, 0)),
            pl.BlockSpec((1, C, S), lambda i: (i, 0, 0)),
        ],
        out_shape=[
            jax.ShapeDtypeStruct((B * K, 8, S), jnp.float32),
            jax.ShapeDtypeStruct((B * K, C, S), jnp.float32),
        ],
    )(cropped, W2t, b2, Wheadt)
    return head, centered


def kernel(points, labels, W1, b1, Wsem1, Woff1, Wmask1, W2, b2, Wsem2, Woff2, Wmask2):
    Whead1t = jnp.pad(jnp.concatenate([Wsem1, Woff1, Wmask1], axis=1),
                      ((0, 0), (0, 2))).T
    Whead2t = jnp.pad(jnp.concatenate([Wsem2, Woff2, Wmask2], axis=1),
                      ((0, 0), (0, 2))).T

    points_p = jnp.pad(points, ((0, 0), (0, 0), (0, NP - N)))
    head1 = _stage1(points_p, W1.T, b1[:, None], Whead1t)
    sem1 = jnp.swapaxes(head1[:, :K, :N], 1, 2)
    off1 = jnp.swapaxes(head1[:, K:K + 3, :N], 1, 2)
    mask1 = jnp.swapaxes(head1[:, K + 3:K + 4, :N], 1, 2)

    coords = jnp.swapaxes(points[:, :3, :], 1, 2)
    lab = labels[:, 0, :]

    def centroids_b(cb, lb):
        s = jax.ops.segment_sum(cb, lb, num_segments=K)
        cnt = jax.ops.segment_sum(jnp.ones((cb.shape[0],), jnp.float32), lb,
                                  num_segments=K)
        return s / jnp.maximum(cnt, 1.0)[:, None]

    cents = jax.vmap(centroids_b)(coords, lab)

    d2 = jnp.sum((coords[:, None, :, :] - cents[:, :, None, :]) ** 2, axis=-1)
    _, idx = jax.lax.top_k(-d2, S)

    pts_e = jnp.broadcast_to(points[:, None, :, :], (B, K, C, N))
    idx_e = jnp.broadcast_to(idx[:, :, None, :], (B, K, C, S))
    cropped = jnp.take_along_axis(pts_e, idx_e, axis=3)

    head2, centered = _stage2(cropped.reshape(B * K, C, S), W2.T, b2[:, None],
                              Whead2t)
    sem2 = jnp.swapaxes(head2[:, :2, :], 1, 2)
    off2 = jnp.swapaxes(head2[:, 2:5, :], 1, 2)
    mask2 = jnp.swapaxes(head2[:, 5:6, :], 1, 2)
    centered = centered.reshape(B, K, C, S)
    return (sem1, off1, mask1, sem2, off2, mask2, centered)


# speedup vs baseline: 1.1717x; 1.0792x over previous
import functools

import jax
import jax.numpy as jnp
from jax.experimental import pallas as pl

B, C, N = 2, 6, 24000
K = 10
S = 3072
H = 256

NP = 24576
BLK1 = 3072


def _stage1_body(pts_ref, w1t_ref, b1_ref, wht_ref, head_ref):
    x = pts_ref[0]
    feat = jax.nn.relu(
        jax.lax.dot_general(w1t_ref[...], x, (((1,), (0,)), ((), ())),
                            preferred_element_type=jnp.float32)
        + b1_ref[...]
    )
    head_ref[0] = jax.lax.dot_general(wht_ref[...], feat,
                                      (((1,), (0,)), ((), ())),
                                      preferred_element_type=jnp.float32)


def _stage1(points, W1t, b1, Wheadt):
    nb = NP // BLK1
    head = pl.pallas_call(
        _stage1_body,
        grid=(B, nb),
        in_specs=[
            pl.BlockSpec((1, C, BLK1), lambda b, j: (b, 0, j)),
            pl.BlockSpec((H, C), lambda b, j: (0, 0)),
            pl.BlockSpec((H, 1), lambda b, j: (0, 0)),
            pl.BlockSpec((16, H), lambda b, j: (0, 0)),
        ],
        out_specs=pl.BlockSpec((1, 16, BLK1), lambda b, j: (b, 0, j)),
        out_shape=jax.ShapeDtypeStruct((B, 16, NP), jnp.float32),
    )(points, W1t, b1, Wheadt)
    return head


def _stage2_body(crop_ref, w2t_ref, b2_ref, wht_ref, head_ref, cent_ref):
    x = crop_ref[0]
    xyz = x[:3, :]
    mean = jnp.sum(xyz, axis=1, keepdims=True) / S
    ctr = jnp.concatenate([xyz - mean, x[3:, :]], axis=0)
    feat = jax.nn.relu(
        jax.lax.dot_general(w2t_ref[...], ctr, (((1,), (0,)), ((), ())),
                            preferred_element_type=jnp.float32)
        + b2_ref[...]
    )
    head_ref[0] = jax.lax.dot_general(wht_ref[...], feat,
                                      (((1,), (0,)), ((), ())),
                                      preferred_element_type=jnp.float32)
    cent_ref[0] = ctr


def _stage2(cropped, W2t, b2, Wheadt):
    head, centered = pl.pallas_call(
        _stage2_body,
        grid=(B * K,),
        in_specs=[
            pl.BlockSpec((1, C, S), lambda i: (i, 0, 0)),
            pl.BlockSpec((H, C), lambda i: (0, 0)),
            pl.BlockSpec((H, 1), lambda i: (0, 0)),
            pl.BlockSpec((8, H), lambda i: (0, 0)),
        ],
        out_specs=[
            pl.BlockSpec((1, 8, S), lambda i: (i, 0, 0)),
            pl.BlockSpec((1, C, S), lambda i: (i, 0, 0)),
        ],
        out_shape=[
            jax.ShapeDtypeStruct((B * K, 8, S), jnp.float32),
            jax.ShapeDtypeStruct((B * K, C, S), jnp.float32),
        ],
    )(cropped, W2t, b2, Wheadt)
    return head, centered


def kernel(points, labels, W1, b1, Wsem1, Woff1, Wmask1, W2, b2, Wsem2, Woff2, Wmask2):
    Whead1t = jnp.pad(jnp.concatenate([Wsem1, Woff1, Wmask1], axis=1),
                      ((0, 0), (0, 2))).T
    Whead2t = jnp.pad(jnp.concatenate([Wsem2, Woff2, Wmask2], axis=1),
                      ((0, 0), (0, 2))).T

    points_p = jnp.pad(points, ((0, 0), (0, 0), (0, NP - N)))
    head1 = _stage1(points_p, W1.T, b1[:, None], Whead1t)
    sem1 = jnp.swapaxes(head1[:, :K, :N], 1, 2)
    off1 = jnp.swapaxes(head1[:, K:K + 3, :N], 1, 2)
    mask1 = jnp.swapaxes(head1[:, K + 3:K + 4, :N], 1, 2)

    coords = jnp.swapaxes(points[:, :3, :], 1, 2)
    lab = labels[:, 0, :]

    def centroids_b(cb, lb):
        s = jax.ops.segment_sum(cb, lb, num_segments=K)
        cnt = jax.ops.segment_sum(jnp.ones((cb.shape[0],), jnp.float32), lb,
                                  num_segments=K)
        return s / jnp.maximum(cnt, 1.0)[:, None]

    cents = jax.vmap(centroids_b)(coords, lab)

    d2 = jnp.sum((coords[:, None, :, :] - cents[:, :, None, :]) ** 2, axis=-1)
    keys = jax.lax.bitcast_convert_type(d2, jnp.int32)
    _, idx = jax.lax.top_k(-keys, S)

    pts_e = jnp.broadcast_to(points[:, None, :, :], (B, K, C, N))
    idx_e = jnp.broadcast_to(idx[:, :, None, :], (B, K, C, S))
    cropped = jnp.take_along_axis(pts_e, idx_e, axis=3)

    head2, centered = _stage2(cropped.reshape(B * K, C, S), W2.T, b2[:, None],
                              Whead2t)
    sem2 = jnp.swapaxes(head2[:, :2, :], 1, 2)
    off2 = jnp.swapaxes(head2[:, 2:5, :], 1, 2)
    mask2 = jnp.swapaxes(head2[:, 5:6, :], 1, 2)
    centered = centered.reshape(B, K, C, S)
    return (sem1, off1, mask1, sem2, off2, mask2, centered)
